# baseline (speedup 1.0000x reference)
import jax
import jax.numpy as jnp
import numpy as np
from jax.experimental import pallas as pl
from jax.experimental.pallas import tpu as pltpu

_Z_MIN = 0.5
_Z_MAX = 32.0
_DELTA = 0.5


def _interp_matrix(S):
    depth_steps = jnp.arange(_Z_MIN, _Z_MAX + _DELTA, _DELTA)
    log_steps = jnp.log2(depth_steps)
    log_min = np.log2(_Z_MIN)
    log_max = np.log2(_Z_MAX)
    norm = (log_steps - log_min) / (log_max - log_min)
    pos = norm * (S - 1)
    i0 = jnp.clip(jnp.floor(pos).astype(jnp.int32), 0, S - 1)
    i1 = jnp.clip(i0 + 1, 0, S - 1)
    w = pos - i0.astype(pos.dtype)
    Z = pos.shape[0]
    zi = jnp.arange(Z)
    M = jnp.zeros((S, Z), jnp.float32)
    M = M.at[i0, zi].add(1.0 - w)
    M = M.at[i1, zi].add(w)
    return M


def _body(img_ref, pld_ref, m_ref, out_ref, cell_ref, pt_ref):
    _, D, H, W = img_ref.shape
    S = pld_ref.shape[-1]
    Z = m_ref.shape[-1]

    pld = pld_ref[0].reshape(H * W, S)
    scores = jnp.dot(pld, m_ref[...], preferred_element_type=jnp.float32)
    s3 = scores.reshape(H, W, Z)

    mx = jnp.max(s3, axis=0)
    e = jnp.exp(s3 - mx[None])
    ssum = jnp.sum(e, axis=0)
    cell_ref[0] = jnp.log(ssum) + mx
    prob = e * (1.0 / ssum)[None]
    pt_ref[...] = jnp.transpose(prob, (0, 2, 1))

    def d_step(d, carry):
        slab = img_ref[0, d]
        acc = jnp.zeros((Z, W), jnp.float32)
        for h in range(H):
            acc = acc + slab[h:h + 1, :] * pt_ref[h]
        out_ref[0, d] = acc
        return carry

    jax.lax.fori_loop(0, D, d_step, 0)


def kernel(image, polar_log_depths):
    B, D, H, W = image.shape
    S = polar_log_depths.shape[-1]
    m = _interp_matrix(S)
    Z = m.shape[1]

    out, cell = pl.pallas_call(
        _body,
        grid=(B,),
        in_specs=[
            pl.BlockSpec((1, D, H, W), lambda b: (b, 0, 0, 0)),
            pl.BlockSpec((1, H, W, S), lambda b: (b, 0, 0, 0)),
            pl.BlockSpec((S, Z), lambda b: (0, 0)),
        ],
        out_specs=[
            pl.BlockSpec((1, D, Z, W), lambda b: (b, 0, 0, 0)),
            pl.BlockSpec((1, W, Z), lambda b: (b, 0, 0)),
        ],
        out_shape=[
            jax.ShapeDtypeStruct((B, D, Z, W), jnp.float32),
            jax.ShapeDtypeStruct((B, W, Z), jnp.float32),
        ],
        scratch_shapes=[pltpu.VMEM((H, Z, W), jnp.float32)],
        compiler_params=pltpu.CompilerParams(
            dimension_semantics=("parallel",),
            vmem_limit_bytes=40 * 1024 * 1024,
        ),
        name="polar_projection_depth",
    )(image, polar_log_depths, m)
    return out, cell


# speedup vs baseline: 1.0276x; 1.0276x over previous
import jax
import jax.numpy as jnp
import numpy as np
from jax.experimental import pallas as pl
from jax.experimental.pallas import tpu as pltpu

_Z_MIN = 0.5
_Z_MAX = 32.0
_DELTA = 0.5
_WG = 4


def _interp_matrix(S):
    depth_steps = jnp.arange(_Z_MIN, _Z_MAX + _DELTA, _DELTA)
    log_steps = jnp.log2(depth_steps)
    log_min = np.log2(_Z_MIN)
    log_max = np.log2(_Z_MAX)
    norm = (log_steps - log_min) / (log_max - log_min)
    pos = norm * (S - 1)
    i0 = jnp.clip(jnp.floor(pos).astype(jnp.int32), 0, S - 1)
    i1 = jnp.clip(i0 + 1, 0, S - 1)
    w = pos - i0.astype(pos.dtype)
    Z = pos.shape[0]
    zi = jnp.arange(Z)
    M = jnp.zeros((S, Z), jnp.float32)
    M = M.at[i0, zi].add(1.0 - w)
    M = M.at[i1, zi].add(w)
    return M


def _body(img_ref, pld_ref, m_ref, out_ref, cell_ref, p_ref, it_ref, o_ref):
    _, D, HW = img_ref.shape
    S = pld_ref.shape[-1]
    Z = m_ref.shape[-1]
    H, W, _ = p_ref.shape
    G = _WG

    pld = pld_ref[0].reshape(HW, S)
    scores = jnp.dot(pld, m_ref[...], preferred_element_type=jnp.float32)
    s3 = scores.reshape(H, W, Z)

    mx = jnp.max(s3, axis=0)
    e = jnp.exp(s3 - mx[None])
    ssum = jnp.sum(e, axis=0)
    cell_ref[0] = jnp.log(ssum) + mx
    p_ref[...] = e * (1.0 / ssum)[None]

    it_ref[...] = img_ref[0].T.reshape(H, W, D)

    K = G * H
    lane = jax.lax.broadcasted_iota(jnp.int32, (K, Z * G), 1)
    row = jax.lax.broadcasted_iota(jnp.int32, (K, Z * G), 0)
    maskc = (row % G == lane % G).astype(jnp.float32)
    gather_idx = (jnp.arange(Z * G, dtype=jnp.int32) // G)[None, :]

    for g in range(W // G):
        rr = p_ref[:, g * G:(g + 1) * G, :].reshape(K, Z)
        r_int = jnp.take_along_axis(
            rr, jnp.broadcast_to(gather_idx, (K, Z * G)), axis=1)
        r_blk = r_int * maskc
        a_g = it_ref[:, g * G:(g + 1) * G, :].reshape(K, D)
        c_g = jax.lax.dot_general(
            r_blk, a_g, (((0,), (0,)), ((), ())),
            preferred_element_type=jnp.float32)
        o_ref[:, g * G:(g + 1) * G, :] = c_g.reshape(Z, G, D)

    out_ref[0] = o_ref[...].reshape(Z * W, D).T


def kernel(image, polar_log_depths):
    B, D, H, W = image.shape
    S = polar_log_depths.shape[-1]
    m = _interp_matrix(S)
    Z = m.shape[1]

    out, cell = pl.pallas_call(
        _body,
        grid=(B,),
        in_specs=[
            pl.BlockSpec((1, D, H * W), lambda b: (b, 0, 0)),
            pl.BlockSpec((1, H, W, S), lambda b: (b, 0, 0, 0)),
            pl.BlockSpec((S, Z), lambda b: (0, 0)),
        ],
        out_specs=[
            pl.BlockSpec((1, D, Z * W), lambda b: (b, 0, 0)),
            pl.BlockSpec((1, W, Z), lambda b: (b, 0, 0)),
        ],
        out_shape=[
            jax.ShapeDtypeStruct((B, D, Z * W), jnp.float32),
            jax.ShapeDtypeStruct((B, W, Z), jnp.float32),
        ],
        scratch_shapes=[
            pltpu.VMEM((H, W, Z), jnp.float32),
            pltpu.VMEM((H, W, D), jnp.float32),
            pltpu.VMEM((Z, W, D), jnp.float32),
        ],
        compiler_params=pltpu.CompilerParams(
            dimension_semantics=("parallel",),
            vmem_limit_bytes=48 * 1024 * 1024,
        ),
        name="polar_projection_depth",
    )(image.reshape(B, D, H * W), polar_log_depths, m)
    return out.reshape(B, D, Z, W), cell


# speedup vs baseline: 1.0810x; 1.0520x over previous
import jax
import jax.numpy as jnp
import numpy as np
from jax.experimental import pallas as pl
from jax.experimental.pallas import tpu as pltpu

_Z_MIN = 0.5
_Z_MAX = 32.0
_DELTA = 0.5
_WG = 4


def _interp_matrix(S):
    depth_steps = np.arange(_Z_MIN, _Z_MAX + _DELTA, _DELTA, dtype=np.float32)
    log_steps = np.log2(depth_steps).astype(np.float32)
    log_min = np.float32(np.log2(_Z_MIN))
    log_max = np.float32(np.log2(_Z_MAX))
    norm = ((log_steps - log_min) / (log_max - log_min)).astype(np.float32)
    pos = (norm * np.float32(S - 1)).astype(np.float32)
    i0 = np.clip(np.floor(pos).astype(np.int32), 0, S - 1)
    i1 = np.clip(i0 + 1, 0, S - 1)
    w = (pos - i0.astype(np.float32)).astype(np.float32)
    Z = pos.shape[0]
    zi = np.arange(Z)
    M = np.zeros((S, Z), np.float32)
    np.add.at(M, (i0, zi), np.float32(1.0) - w)
    np.add.at(M, (i1, zi), w)
    return jnp.asarray(M)


def _body(img_ref, pld_ref, m_ref, out_ref, cell_ref, p_ref, it_ref, o_ref):
    _, D, HW = img_ref.shape
    S = pld_ref.shape[-1]
    Z = m_ref.shape[-1]
    H, W, _ = p_ref.shape
    G = _WG

    pld = pld_ref[0].reshape(HW, S)
    scores = jnp.dot(pld, m_ref[...], preferred_element_type=jnp.float32)
    s3 = scores.reshape(H, W, Z)

    mx = jnp.max(s3, axis=0)
    e = jnp.exp(s3 - mx[None])
    ssum = jnp.sum(e, axis=0)
    cell_ref[0] = jnp.log(ssum) + mx
    p_ref[...] = e * (1.0 / ssum)[None]

    it_ref[...] = img_ref[0].T.reshape(H, W, D)

    K = G * H
    lane = jax.lax.broadcasted_iota(jnp.int32, (K, Z * G), 1)
    row = jax.lax.broadcasted_iota(jnp.int32, (K, Z * G), 0)
    maskc = (row % G == lane % G).astype(jnp.float32)
    gather_idx = (jnp.arange(Z * G, dtype=jnp.int32) // G)[None, :]

    for g in range(W // G):
        rr = p_ref[:, g * G:(g + 1) * G, :].reshape(K, Z)
        r_int = jnp.take_along_axis(
            rr, jnp.broadcast_to(gather_idx, (K, Z * G)), axis=1)
        r_blk = r_int * maskc
        a_g = it_ref[:, g * G:(g + 1) * G, :].reshape(K, D)
        c_g = jax.lax.dot_general(
            r_blk, a_g, (((0,), (0,)), ((), ())),
            preferred_element_type=jnp.float32)
        o_ref[:, g * G:(g + 1) * G, :] = c_g.reshape(Z, G, D)

    out_ref[0] = o_ref[...].reshape(Z * W, D).T


def kernel(image, polar_log_depths):
    B, D, H, W = image.shape
    S = polar_log_depths.shape[-1]
    m = _interp_matrix(S)
    Z = m.shape[1]

    out, cell = pl.pallas_call(
        _body,
        grid=(B,),
        in_specs=[
            pl.BlockSpec((1, D, H * W), lambda b: (b, 0, 0)),
            pl.BlockSpec((1, H, W, S), lambda b: (b, 0, 0, 0)),
            pl.BlockSpec((S, Z), lambda b: (0, 0)),
        ],
        out_specs=[
            pl.BlockSpec((1, D, Z * W), lambda b: (b, 0, 0)),
            pl.BlockSpec((1, W, Z), lambda b: (b, 0, 0)),
        ],
        out_shape=[
            jax.ShapeDtypeStruct((B, D, Z * W), jnp.float32),
            jax.ShapeDtypeStruct((B, W, Z), jnp.float32),
        ],
        scratch_shapes=[
            pltpu.VMEM((H, W, Z), jnp.float32),
            pltpu.VMEM((H, W, D), jnp.float32),
            pltpu.VMEM((Z, W, D), jnp.float32),
        ],
        compiler_params=pltpu.CompilerParams(
            dimension_semantics=("parallel",),
            vmem_limit_bytes=48 * 1024 * 1024,
        ),
        name="polar_projection_depth",
    )(image.reshape(B, D, H * W), polar_log_depths, m)
    return out.reshape(B, D, Z, W), cell


# speedup vs baseline: 1.2611x; 1.1666x over previous
import jax
import jax.numpy as jnp
import numpy as np
from jax.experimental import pallas as pl
from jax.experimental.pallas import tpu as pltpu

_Z_MIN = 0.5
_Z_MAX = 32.0
_DELTA = 0.5
_WG = 4


def _interp_matrix(S):
    depth_steps = np.arange(_Z_MIN, _Z_MAX + _DELTA, _DELTA, dtype=np.float32)
    log_steps = np.log2(depth_steps).astype(np.float32)
    log_min = np.float32(np.log2(_Z_MIN))
    log_max = np.float32(np.log2(_Z_MAX))
    norm = ((log_steps - log_min) / (log_max - log_min)).astype(np.float32)
    pos = (norm * np.float32(S - 1)).astype(np.float32)
    i0 = np.clip(np.floor(pos).astype(np.int32), 0, S - 1)
    i1 = np.clip(i0 + 1, 0, S - 1)
    w = (pos - i0.astype(np.float32)).astype(np.float32)
    Z = pos.shape[0]
    zi = np.arange(Z)
    M = np.zeros((S, Z), np.float32)
    np.add.at(M, (i0, zi), np.float32(1.0) - w)
    np.add.at(M, (i1, zi), w)
    return jnp.asarray(M)


def _body(img_ref, pld_ref, m_ref, out_ref, cell_ref, p_ref, it_ref, o_ref):
    _, D, _, _ = img_ref.shape
    S = pld_ref.shape[-1]
    Z = m_ref.shape[-1]
    H, W, _ = p_ref.shape
    G = _WG

    pld = pld_ref[0].reshape(H * W, S)
    scores = jnp.dot(pld, m_ref[...], preferred_element_type=jnp.float32)
    s3 = scores.reshape(H, W, Z)

    mx = jnp.max(s3, axis=0)
    e = jnp.exp(s3 - mx[None])
    ssum = jnp.sum(e, axis=0)
    cell_ref[0] = jnp.log(ssum) + mx
    p_ref[...] = e * (1.0 / ssum)[None]

    it_ref[...] = jnp.transpose(img_ref[0], (1, 2, 0))

    K = G * H
    lane = jax.lax.broadcasted_iota(jnp.int32, (K, Z * G), 1)
    row = jax.lax.broadcasted_iota(jnp.int32, (K, Z * G), 0)
    maskc = (row % G == lane % G).astype(jnp.float32)
    gather_idx = (jnp.arange(Z * G, dtype=jnp.int32) // G)[None, :]

    for g in range(W // G):
        rr = p_ref[:, g * G:(g + 1) * G, :].reshape(K, Z)
        r_int = jnp.take_along_axis(
            rr, jnp.broadcast_to(gather_idx, (K, Z * G)), axis=1)
        r_blk = r_int * maskc
        a_g = it_ref[:, g * G:(g + 1) * G, :].reshape(K, D)
        c_g = jax.lax.dot_general(
            r_blk, a_g, (((0,), (0,)), ((), ())),
            preferred_element_type=jnp.float32)
        o_ref[:, g * G:(g + 1) * G, :] = c_g.reshape(Z, G, D)

    out_ref[0] = jnp.transpose(o_ref[...], (2, 0, 1))


def kernel(image, polar_log_depths):
    B, D, H, W = image.shape
    S = polar_log_depths.shape[-1]
    m = _interp_matrix(S)
    Z = m.shape[1]

    out, cell = pl.pallas_call(
        _body,
        grid=(B,),
        in_specs=[
            pl.BlockSpec((1, D, H, W), lambda b: (b, 0, 0, 0)),
            pl.BlockSpec((1, H, W, S), lambda b: (b, 0, 0, 0)),
            pl.BlockSpec((S, Z), lambda b: (0, 0)),
        ],
        out_specs=[
            pl.BlockSpec((1, D, Z, W), lambda b: (b, 0, 0, 0)),
            pl.BlockSpec((1, W, Z), lambda b: (b, 0, 0)),
        ],
        out_shape=[
            jax.ShapeDtypeStruct((B, D, Z, W), jnp.float32),
            jax.ShapeDtypeStruct((B, W, Z), jnp.float32),
        ],
        scratch_shapes=[
            pltpu.VMEM((H, W, Z), jnp.float32),
            pltpu.VMEM((H, W, D), jnp.float32),
            pltpu.VMEM((Z, W, D), jnp.float32),
        ],
        compiler_params=pltpu.CompilerParams(
            dimension_semantics=("parallel",),
            vmem_limit_bytes=48 * 1024 * 1024,
        ),
        name="polar_projection_depth",
    )(image, polar_log_depths, m)
    return out, cell


# speedup vs baseline: 1.5948x; 1.2646x over previous
import jax
import jax.numpy as jnp
import numpy as np
from jax.experimental import pallas as pl
from jax.experimental.pallas import tpu as pltpu

_Z_MIN = 0.5
_Z_MAX = 32.0
_DELTA = 0.5
_WG = 4


def _interp_matrix(S):
    depth_steps = np.arange(_Z_MIN, _Z_MAX + _DELTA, _DELTA, dtype=np.float32)
    log_steps = np.log2(depth_steps).astype(np.float32)
    log_min = np.float32(np.log2(_Z_MIN))
    log_max = np.float32(np.log2(_Z_MAX))
    norm = ((log_steps - log_min) / (log_max - log_min)).astype(np.float32)
    pos = (norm * np.float32(S - 1)).astype(np.float32)
    i0 = np.clip(np.floor(pos).astype(np.int32), 0, S - 1)
    i1 = np.clip(i0 + 1, 0, S - 1)
    w = (pos - i0.astype(np.float32)).astype(np.float32)
    Z = pos.shape[0]
    zi = np.arange(Z)
    M = np.zeros((S, Z), np.float32)
    np.add.at(M, (i0, zi), np.float32(1.0) - w)
    np.add.at(M, (i1, zi), w)
    return jnp.asarray(M)


def _body(img_ref, pld_ref, m_ref, out_ref, cell_ref, p_ref, it_ref, o_ref):
    _, D, H, W = img_ref.shape
    S = pld_ref.shape[-1]
    Z = m_ref.shape[-1]
    G = _WG

    pld = pld_ref[0].reshape(H * W, S)
    scores = jnp.dot(pld, m_ref[...], preferred_element_type=jnp.float32)
    s3 = scores.reshape(H, W, Z)

    mx = jnp.max(s3, axis=0)
    e = jnp.exp(s3 - mx[None])
    ssum = jnp.sum(e, axis=0)
    cell_ref[0] = jnp.log(ssum) + mx
    p_ref[...] = e * (1.0 / ssum)[None]

    it_ref[...] = jnp.transpose(img_ref[0], (0, 2, 1))

    K = G * H
    lane = jax.lax.broadcasted_iota(jnp.int32, (D * G, K), 1)
    row = jax.lax.broadcasted_iota(jnp.int32, (D * G, K), 0)
    maskc = (row % G == lane % G).astype(jnp.float32)
    gidx = jnp.broadcast_to(
        (jnp.arange(K, dtype=jnp.int32) // G)[None, :], (D * G, K))

    for g in range(W // G):
        m1 = it_ref[:, g * G:(g + 1) * G, :].reshape(D * G, H)
        l_blk = jnp.take_along_axis(m1, gidx, axis=1) * maskc
        rr = p_ref[:, g * G:(g + 1) * G, :].reshape(K, Z)
        c_g = jax.lax.dot_general(
            l_blk, rr, (((1,), (0,)), ((), ())),
            preferred_element_type=jnp.float32)
        o_ref[:, g * G:(g + 1) * G, :] = c_g.reshape(D, G, Z)

    out_ref[0] = jnp.transpose(o_ref[...], (0, 2, 1))


def kernel(image, polar_log_depths):
    B, D, H, W = image.shape
    S = polar_log_depths.shape[-1]
    m = _interp_matrix(S)
    Z = m.shape[1]

    out, cell = pl.pallas_call(
        _body,
        grid=(B,),
        in_specs=[
            pl.BlockSpec((1, D, H, W), lambda b: (b, 0, 0, 0)),
            pl.BlockSpec((1, H, W, S), lambda b: (b, 0, 0, 0)),
            pl.BlockSpec((S, Z), lambda b: (0, 0)),
        ],
        out_specs=[
            pl.BlockSpec((1, D, Z, W), lambda b: (b, 0, 0, 0)),
            pl.BlockSpec((1, W, Z), lambda b: (b, 0, 0)),
        ],
        out_shape=[
            jax.ShapeDtypeStruct((B, D, Z, W), jnp.float32),
            jax.ShapeDtypeStruct((B, W, Z), jnp.float32),
        ],
        scratch_shapes=[
            pltpu.VMEM((H, W, Z), jnp.float32),
            pltpu.VMEM((D, W, H), jnp.float32),
            pltpu.VMEM((D, W, Z), jnp.float32),
        ],
        compiler_params=pltpu.CompilerParams(
            dimension_semantics=("parallel",),
            vmem_limit_bytes=48 * 1024 * 1024,
        ),
        name="polar_projection_depth",
    )(image, polar_log_depths, m)
    return out, cell


# speedup vs baseline: 1.7534x; 1.0994x over previous
import jax
import jax.numpy as jnp
import numpy as np
from jax.experimental import pallas as pl
from jax.experimental.pallas import tpu as pltpu

_Z_MIN = 0.5
_Z_MAX = 32.0
_DELTA = 0.5
_WG = 4


def _interp_matrix(S):
    depth_steps = np.arange(_Z_MIN, _Z_MAX + _DELTA, _DELTA, dtype=np.float32)
    log_steps = np.log2(depth_steps).astype(np.float32)
    log_min = np.float32(np.log2(_Z_MIN))
    log_max = np.float32(np.log2(_Z_MAX))
    norm = ((log_steps - log_min) / (log_max - log_min)).astype(np.float32)
    pos = (norm * np.float32(S - 1)).astype(np.float32)
    i0 = np.clip(np.floor(pos).astype(np.int32), 0, S - 1)
    i1 = np.clip(i0 + 1, 0, S - 1)
    w = (pos - i0.astype(np.float32)).astype(np.float32)
    Z = pos.shape[0]
    zi = np.arange(Z)
    M = np.zeros((S, Z), np.float32)
    np.add.at(M, (i0, zi), np.float32(1.0) - w)
    np.add.at(M, (i1, zi), w)
    return jnp.asarray(M)


def _body(img_ref, pld_ref, m_ref, out_ref, cell_ref, p_ref, it_ref, o_ref):
    _, D, H, W = img_ref.shape
    S = pld_ref.shape[-1]
    Z = m_ref.shape[-1]
    G = _WG

    pld = pld_ref[0].reshape(H * W, S)
    scores = jnp.dot(pld, m_ref[...], preferred_element_type=jnp.float32)
    s3 = scores.reshape(H, W, Z)

    mx = jnp.max(s3, axis=0)
    e = jnp.exp(s3 - mx[None])
    ssum = jnp.sum(e, axis=0)
    cell_ref[0] = jnp.log(ssum) + mx
    p_ref[...] = e * (1.0 / ssum)[None]

    it_ref[...] = jnp.transpose(img_ref[0], (0, 2, 1))

    K = G * H
    lane = jax.lax.broadcasted_iota(jnp.int32, (D * G, K), 1)
    row = jax.lax.broadcasted_iota(jnp.int32, (D * G, K), 0)
    maskc = (row % G == lane % G).astype(jnp.float32)
    hrow = jax.lax.broadcasted_iota(jnp.int32, (H, K), 0)
    hcol = jax.lax.broadcasted_iota(jnp.int32, (H, K), 1)
    spread = (hcol // G == hrow).astype(jnp.float32)

    for g in range(W // G):
        m1 = it_ref[:, g * G:(g + 1) * G, :].reshape(D * G, H)
        l_blk = jnp.dot(m1, spread,
                        preferred_element_type=jnp.float32) * maskc
        rr = p_ref[:, g * G:(g + 1) * G, :].reshape(K, Z)
        c_g = jax.lax.dot_general(
            l_blk, rr, (((1,), (0,)), ((), ())),
            preferred_element_type=jnp.float32)
        o_ref[:, g * G:(g + 1) * G, :] = c_g.reshape(D, G, Z)

    out_ref[0] = jnp.transpose(o_ref[...], (0, 2, 1))


def kernel(image, polar_log_depths):
    B, D, H, W = image.shape
    S = polar_log_depths.shape[-1]
    m = _interp_matrix(S)
    Z = m.shape[1]

    out, cell = pl.pallas_call(
        _body,
        grid=(B,),
        in_specs=[
            pl.BlockSpec((1, D, H, W), lambda b: (b, 0, 0, 0)),
            pl.BlockSpec((1, H, W, S), lambda b: (b, 0, 0, 0)),
            pl.BlockSpec((S, Z), lambda b: (0, 0)),
        ],
        out_specs=[
            pl.BlockSpec((1, D, Z, W), lambda b: (b, 0, 0, 0)),
            pl.BlockSpec((1, W, Z), lambda b: (b, 0, 0)),
        ],
        out_shape=[
            jax.ShapeDtypeStruct((B, D, Z, W), jnp.float32),
            jax.ShapeDtypeStruct((B, W, Z), jnp.float32),
        ],
        scratch_shapes=[
            pltpu.VMEM((H, W, Z), jnp.float32),
            pltpu.VMEM((D, W, H), jnp.float32),
            pltpu.VMEM((D, W, Z), jnp.float32),
        ],
        compiler_params=pltpu.CompilerParams(
            dimension_semantics=("parallel",),
            vmem_limit_bytes=48 * 1024 * 1024,
        ),
        name="polar_projection_depth",
    )(image, polar_log_depths, m)
    return out, cell
